# Initial kernel scaffold; baseline (speedup 1.0000x reference)
import jax
import jax.numpy as jnp
from jax.experimental import pallas as pl


def kernel(edge_nodes, edge_feats, edge_index, W0, b0):
    raise NotImplementedError("write your pallas kernel here")


# speedup vs baseline: 1.8559x; 1.8559x over previous
import functools

import jax
import jax.numpy as jnp
from jax import lax
from jax.experimental import pallas as pl
from jax.experimental.pallas import tpu as pltpu
from jax.experimental.pallas import tpu_sc as plsc

_NC = 2
_NS = 16
_CH = 80


def _sc_scatter(src, dst, feats, zer, n_pad):
    E, D = feats.shape
    nw = _NC * _NS
    epw = E // nw
    nch = epw // _CH
    rpt = n_pad // _NS
    rch = rpt // _CH
    mesh = plsc.VectorSubcoreMesh(core_axis_name="c", subcore_axis_name="s")

    @functools.partial(
        pl.kernel,
        mesh=mesh,
        out_type=jax.ShapeDtypeStruct((_NC * n_pad, D), jnp.float32),
        scratch_types=[
            pltpu.VMEM((_CH, D), jnp.float32),
            pltpu.VMEM((_CH,), jnp.int32),
            pltpu.VMEM((_CH,), jnp.int32),
            pltpu.VMEM_SHARED((n_pad, D), jnp.float32),
        ],
    )
    def body(src_h, dst_h, feats_h, zer_h, out_h,
             rows_v, si_v, di_v, g0_sh):
        cid = lax.axis_index("c")
        sid = lax.axis_index("s")
        wid = cid * _NS + sid
        row0 = sid * rpt

        pltpu.sync_copy(zer_h, rows_v)
        for k in range(rch):
            pltpu.sync_copy(rows_v, g0_sh.at[pl.ds(row0 + k * _CH, _CH)])
        plsc.subcore_barrier()

        @pl.loop(0, nch)
        def step(i):
            base = pl.multiple_of(wid * epw + i * _CH, 8)
            pltpu.sync_copy(src_h.at[pl.ds(base, _CH)], si_v)
            pltpu.sync_copy(dst_h.at[pl.ds(base, _CH)], di_v)
            pltpu.sync_copy(feats_h.at[pl.ds(base, _CH)], rows_v)
            pltpu.sync_copy(rows_v, g0_sh.at[si_v], add=True)
            pltpu.sync_copy(rows_v, g0_sh.at[di_v], add=True)

        plsc.subcore_barrier()

        out0 = cid * n_pad + row0
        for k in range(rch):
            pltpu.sync_copy(g0_sh.at[pl.ds(row0 + k * _CH, _CH)], rows_v)
            pltpu.sync_copy(rows_v, out_h.at[pl.ds(out0 + k * _CH, _CH)])

    return body(src, dst, feats, zer)


def _tc_epilogue(en_pad, g0_parts, cnt2d, W0, b0_2d):
    B = en_pad.shape[0]
    K = W0.shape[1]

    def body(en_ref, g_ref, c_ref, w_ref, b_ref, out_ref):
        hi = jax.lax.Precision.HIGHEST
        en = en_ref[...]
        g = g_ref[0] + g_ref[1]
        t = jax.lax.dot(en, g, precision=hi,
                        preferred_element_type=jnp.float32)
        rs = jnp.sum(en * c_ref[...], axis=1, keepdims=True)
        inv = 1.0 / rs
        inv = jnp.where(jnp.isinf(inv), 0.0, inv)
        h = jax.lax.dot(t, w_ref[...], precision=hi,
                        preferred_element_type=jnp.float32)
        out_ref[...] = (h + rs * b_ref[...]) * inv

    return pl.pallas_call(
        body,
        out_shape=jax.ShapeDtypeStruct((B, K), jnp.float32),
    )(en_pad, g0_parts, cnt2d, W0, b0_2d)


def kernel(edge_nodes, edge_feats, edge_index, W0, b0):
    B, N = edge_nodes.shape
    E, D = edge_feats.shape
    K = W0.shape[1]
    step = _NS * _CH
    n_pad = ((N + step - 1) // step) * step

    src = edge_index[0]
    dst = edge_index[1]
    zer = jnp.zeros((_CH, D), jnp.float32)

    g0_flat = _sc_scatter(src, dst, edge_feats, zer, n_pad)
    g0_parts = g0_flat.reshape(_NC, n_pad, D)
    cnt = jnp.zeros((n_pad,), jnp.float32).at[src].add(1.0).at[dst].add(1.0)
    en_pad = jnp.pad(edge_nodes, ((0, 0), (0, n_pad - N)))

    return _tc_epilogue(en_pad, g0_parts, cnt[None, :], W0, b0.reshape(1, K))


# speedup vs baseline: 2.2413x; 1.2077x over previous
import functools

import jax
import jax.numpy as jnp
from jax import lax
from jax.experimental import pallas as pl
from jax.experimental.pallas import tpu as pltpu
from jax.experimental.pallas import tpu_sc as plsc

_NC = 2
_NS = 16
_CH = 80


def _sc_scatter(src, dst, feats, zer, n_pad):
    E, D = feats.shape
    nw = _NC * _NS
    epw = E // nw
    nch = epw // _CH
    rpt = n_pad // _NS
    rch = rpt // _CH
    mesh = plsc.VectorSubcoreMesh(core_axis_name="c", subcore_axis_name="s")

    assert nch % 2 == 1 and nch >= 3
    scratch_types = [
        pltpu.VMEM((_CH, D), jnp.float32),
        pltpu.VMEM((_CH, D), jnp.float32),
        pltpu.VMEM((_CH,), jnp.int32),
        pltpu.VMEM((_CH,), jnp.int32),
        pltpu.VMEM((_CH,), jnp.int32),
        pltpu.VMEM((_CH,), jnp.int32),
        pltpu.VMEM_SHARED((n_pad, D), jnp.float32),
        pltpu.SemaphoreType.DMA,
        pltpu.SemaphoreType.DMA,
    ]

    @functools.partial(
        pl.kernel,
        mesh=mesh,
        out_type=jax.ShapeDtypeStruct((_NC * n_pad, D), jnp.float32),
        scratch_types=scratch_types,
    )
    def body(src_h, dst_h, feats_h, zer_h, out_h,
             rows_a, rows_b, si_a, si_b, di_a, di_b, g0_sh, sem_a, sem_b):
        cid = lax.axis_index("c")
        sid = lax.axis_index("s")
        wid = cid * _NS + sid
        row0 = sid * rpt

        def start(i, rows_v, si_v, di_v, sem):
            base = pl.multiple_of(wid * epw + i * _CH, 8)
            pltpu.async_copy(src_h.at[pl.ds(base, _CH)], si_v, sem)
            pltpu.async_copy(dst_h.at[pl.ds(base, _CH)], di_v, sem)
            pltpu.async_copy(feats_h.at[pl.ds(base, _CH)], rows_v, sem)

        def wait(rows_v, si_v, di_v, sem):
            pltpu.make_async_copy(src_h.at[pl.ds(0, _CH)], si_v, sem).wait()
            pltpu.make_async_copy(dst_h.at[pl.ds(0, _CH)], di_v, sem).wait()
            pltpu.make_async_copy(feats_h.at[pl.ds(0, _CH)], rows_v,
                                  sem).wait()

        def scatter(rows_v, si_v, di_v):
            pltpu.sync_copy(rows_v, g0_sh.at[si_v], add=True)
            pltpu.sync_copy(rows_v, g0_sh.at[di_v], add=True)

        pltpu.sync_copy(zer_h, rows_a)
        for k in range(rch):
            pltpu.sync_copy(rows_a, g0_sh.at[pl.ds(row0 + k * _CH, _CH)])
        plsc.subcore_barrier()

        start(0, rows_a, si_a, di_a, sem_a)

        @pl.loop(0, nch // 2)
        def step(p):
            e = 2 * p
            wait(rows_a, si_a, di_a, sem_a)
            start(e + 1, rows_b, si_b, di_b, sem_b)
            scatter(rows_a, si_a, di_a)
            wait(rows_b, si_b, di_b, sem_b)
            start(e + 2, rows_a, si_a, di_a, sem_a)
            scatter(rows_b, si_b, di_b)

        wait(rows_a, si_a, di_a, sem_a)
        scatter(rows_a, si_a, di_a)

        plsc.subcore_barrier()

        out0 = cid * n_pad + row0
        for k in range(rch):
            pltpu.sync_copy(g0_sh.at[pl.ds(row0 + k * _CH, _CH)], rows_a)
            pltpu.sync_copy(rows_a, out_h.at[pl.ds(out0 + k * _CH, _CH)])

    return body(src, dst, feats, zer)


def _tc_epilogue(en_pad, g0_parts, cnt2d, W0, b0_2d):
    B = en_pad.shape[0]
    K = W0.shape[1]

    def body(en_ref, g_ref, c_ref, w_ref, b_ref, out_ref):
        hi = jax.lax.Precision.HIGHEST
        en = en_ref[...]
        g = g_ref[0] + g_ref[1]
        t = jax.lax.dot(en, g, precision=hi,
                        preferred_element_type=jnp.float32)
        rs = jnp.sum(en * c_ref[...], axis=1, keepdims=True)
        inv = 1.0 / rs
        inv = jnp.where(jnp.isinf(inv), 0.0, inv)
        h = jax.lax.dot(t, w_ref[...], precision=hi,
                        preferred_element_type=jnp.float32)
        out_ref[...] = (h + rs * b_ref[...]) * inv

    return pl.pallas_call(
        body,
        out_shape=jax.ShapeDtypeStruct((B, K), jnp.float32),
    )(en_pad, g0_parts, cnt2d, W0, b0_2d)


def kernel(edge_nodes, edge_feats, edge_index, W0, b0):
    B, N = edge_nodes.shape
    E, D = edge_feats.shape
    K = W0.shape[1]
    step = _NS * _CH
    n_pad = ((N + step - 1) // step) * step

    src = edge_index[0]
    dst = edge_index[1]
    zer = jnp.zeros((_CH, D), jnp.float32)

    g0_flat = _sc_scatter(src, dst, edge_feats, zer, n_pad)
    g0_parts = g0_flat.reshape(_NC, n_pad, D)
    ids = jnp.concatenate([src, dst])
    cnt = jnp.zeros((n_pad,), jnp.float32).at[ids].add(1.0)
    en_pad = jnp.pad(edge_nodes, ((0, 0), (0, n_pad - N)))

    return _tc_epilogue(en_pad, g0_parts, cnt[None, :], W0, b0.reshape(1, K))


# speedup vs baseline: 7.8301x; 3.4936x over previous
import functools

import jax
import jax.numpy as jnp
from jax import lax
from jax.experimental import pallas as pl
from jax.experimental.pallas import tpu as pltpu
from jax.experimental.pallas import tpu_sc as plsc

_NC = 2
_NS = 16
_CH = 80


def _sc_scatter(src, dst, feats, zer, n_pad):
    E, D = feats.shape
    nw = _NC * _NS
    epw = E // nw
    nch = epw // _CH
    rpt = n_pad // _NS
    rch = rpt // _CH
    mesh = plsc.VectorSubcoreMesh(core_axis_name="c", subcore_axis_name="s")
    assert nch % 2 == 1 and nch >= 3

    scratch_types = [
        pltpu.VMEM((_CH, D), jnp.float32),
        pltpu.VMEM((_CH, D), jnp.float32),
        pltpu.VMEM((_CH,), jnp.int32),
        pltpu.VMEM((_CH,), jnp.int32),
        pltpu.VMEM((_CH,), jnp.int32),
        pltpu.VMEM((_CH,), jnp.int32),
        pltpu.VMEM_SHARED((n_pad, D), jnp.float32),
        pltpu.SemaphoreType.DMA,
        pltpu.SemaphoreType.DMA,
    ]

    @functools.partial(
        pl.kernel,
        mesh=mesh,
        out_type=jax.ShapeDtypeStruct((_NC * n_pad, D), jnp.float32),
        scratch_types=scratch_types,
    )
    def body(src_h, dst_h, feats_h, zer_h, out_h,
             rows_a, rows_b, si_a, si_b, di_a, di_b, g0_sh, sem_a, sem_b):
        cid = lax.axis_index("c")
        sid = lax.axis_index("s")
        wid = cid * _NS + sid
        row0 = sid * rpt

        def start(i, rows_v, si_v, di_v, sem):
            base = pl.multiple_of(wid * epw + i * _CH, 8)
            pltpu.async_copy(src_h.at[pl.ds(base, _CH)], si_v, sem)
            pltpu.async_copy(dst_h.at[pl.ds(base, _CH)], di_v, sem)
            pltpu.async_copy(feats_h.at[pl.ds(base, _CH)], rows_v, sem)

        def wait(rows_v, si_v, di_v, sem):
            pltpu.make_async_copy(src_h.at[pl.ds(0, _CH)], si_v, sem).wait()
            pltpu.make_async_copy(dst_h.at[pl.ds(0, _CH)], di_v, sem).wait()
            pltpu.make_async_copy(feats_h.at[pl.ds(0, _CH)], rows_v,
                                  sem).wait()

        def scatter(rows_v, si_v, di_v):
            pltpu.sync_copy(rows_v, g0_sh.at[si_v], add=True)
            pltpu.sync_copy(rows_v, g0_sh.at[di_v], add=True)

        pltpu.sync_copy(zer_h, rows_a)
        for k in range(rch):
            pltpu.sync_copy(rows_a, g0_sh.at[pl.ds(row0 + k * _CH, _CH)])
        plsc.subcore_barrier()

        start(0, rows_a, si_a, di_a, sem_a)

        @pl.loop(0, nch // 2)
        def step(p):
            e = 2 * p
            wait(rows_a, si_a, di_a, sem_a)
            start(e + 1, rows_b, si_b, di_b, sem_b)
            scatter(rows_a, si_a, di_a)
            wait(rows_b, si_b, di_b, sem_b)
            start(e + 2, rows_a, si_a, di_a, sem_a)
            scatter(rows_b, si_b, di_b)

        wait(rows_a, si_a, di_a, sem_a)
        scatter(rows_a, si_a, di_a)

        plsc.subcore_barrier()

        out0 = cid * n_pad + row0
        for k in range(rch):
            pltpu.sync_copy(g0_sh.at[pl.ds(row0 + k * _CH, _CH)], rows_a)
            pltpu.sync_copy(rows_a, out_h.at[pl.ds(out0 + k * _CH, _CH)])

    return body(src, dst, feats, zer)


def _tc_epilogue(en_pad, g0_parts, cnt2d, W0, b0_2d):
    B = en_pad.shape[0]
    K = W0.shape[1]

    def body(en_ref, g_ref, c_ref, w_ref, b_ref, out_ref):
        hi = jax.lax.Precision.HIGHEST
        en = en_ref[...]
        g = g_ref[0] + g_ref[1]
        t = jax.lax.dot(en, g, precision=hi,
                        preferred_element_type=jnp.float32)
        rs = jnp.sum(en * c_ref[...], axis=1, keepdims=True)
        inv = 1.0 / rs
        inv = jnp.where(jnp.isinf(inv), 0.0, inv)
        h = jax.lax.dot(t, w_ref[...], precision=hi,
                        preferred_element_type=jnp.float32)
        out_ref[...] = (h + rs * b_ref[...]) * inv

    return pl.pallas_call(
        body,
        out_shape=jax.ShapeDtypeStruct((B, K), jnp.float32),
    )(en_pad, g0_parts, cnt2d, W0, b0_2d)


def kernel(edge_nodes, edge_feats, edge_index, W0, b0):
    B, N = edge_nodes.shape
    E, D = edge_feats.shape
    K = W0.shape[1]
    step = _NS * _CH
    n_pad = ((N + step - 1) // step) * step

    src = edge_index[0]
    dst = edge_index[1]
    zer = jnp.zeros((_CH, D), jnp.float32)

    g0_flat = _sc_scatter(src, dst, edge_feats, zer, n_pad)
    g0_parts = g0_flat.reshape(_NC, n_pad, D)
    ids = jnp.concatenate([src, dst])
    cnt = jnp.zeros((n_pad,), jnp.int32).at[ids].add(1).astype(jnp.float32)
    en_pad = jnp.pad(edge_nodes, ((0, 0), (0, n_pad - N)))

    return _tc_epilogue(en_pad, g0_parts, cnt[None, :], W0, b0.reshape(1, K))
